# baseline (speedup 1.0000x reference)
import functools

import jax
import jax.numpy as jnp
from jax import lax
from jax.experimental import pallas as pl
from jax.experimental.pallas import tpu as pltpu
from jax.experimental.pallas import tpu_sc as plsc

_N = 10000
_D = 128
_HID = 4096
_E_TOT = 170000
_NC = 2
_NS = 16
_NW = _NC * _NS
_B = 32
_NB0 = 234
_NB1 = 105
_NROWS = _NS * (_NB0 + _NB1)
_E_PAD = _NROWS * _B
_RT = 632
_R = 400
_G = _N // _R


def _sc_edge_body(idx_hbm, asrc_hbm, adst_hbm, tab_hbm,
                  agg_hbm, den_hbm,
                  asrc_v, adst_v, den_v, idx_a, idx_b, idx_c,
                  rows_a, rows_b, rows_c, agg_sh,
                  isem_a, isem_b, isem_c, gsem_a, gsem_b, gsem_c,
                  ssem_a, ssem_b, ssem_c):
    cid = lax.axis_index("c")
    sid = lax.axis_index("s")

    pltpu.sync_copy(asrc_hbm, asrc_v)
    pltpu.sync_copy(adst_hbm, adst_v)

    def _zden(i, c):
        den_v[0, pl.ds(i * 16, 16)] = jnp.zeros((16,), jnp.float32)
        return c
    lax.fori_loop(0, _N // 16, _zden, 0)

    def _zrow(i, c):
        for j in range(_D // 16):
            rows_a[i, pl.ds(j * 16, 16)] = jnp.zeros((16,), jnp.float32)
        return c
    lax.fori_loop(0, _B, _zrow, 0)
    base_row = pl.multiple_of(sid * 625 - sid % 8, 8)
    for k in range(_RT // _B):
        pltpu.sync_copy(rows_a, agg_sh.at[pl.ds(base_row + k * _B, _B)])
    pltpu.sync_copy(rows_a, agg_sh.at[pl.ds(base_row + _RT - _B, _B)])
    plsc.subcore_barrier()

    nb = jnp.where(cid == 0, _NB0, _NB1)
    ibase = jnp.where(cid == 0, sid * _NB0, _NS * _NB0 + sid * _NB1)
    ebase = ibase * _B
    wid = cid * _NS + sid

    def _step(b, idx_m, rows_m, gsem_m, isem_m, ssem_m,
              idx_n, rows_n, gsem_n, isem_n, ssem_n):
        pltpu.make_async_copy(tab_hbm.at[idx_m.at[0]], rows_m, gsem_m).wait()

        @pl.when(b + 2 < nb)
        def _():
            pltpu.make_async_copy(idx_hbm.at[ibase + b + 2], idx_n,
                                  isem_n).wait()

            @pl.when(b >= 1)
            def _():
                pltpu.make_async_copy(rows_n, agg_sh.at[idx_n.at[1]],
                                      ssem_n).wait()
            pltpu.async_copy(tab_hbm.at[idx_n.at[0]], rows_n, gsem_n)

        off = ebase + b * _B

        def _grp(g, c2):
            s16 = idx_m[0, pl.ds(g * 16, 16)]
            d16 = idx_m[1, pl.ds(g * 16, 16)]
            av = plsc.load_gather(asrc_v, [s16])
            dv = plsc.load_gather(adst_v, [d16])
            e = av + dv
            e = jnp.maximum(e, 0.2 * e)
            ex = jnp.exp(e)
            gidx = off + g * 16 + lax.iota(jnp.int32, 16)
            ex = jnp.where(gidx < _E_TOT, ex, 0.0)
            idx_m[2, pl.ds(g * 16, 16)] = plsc.bitcast(ex, jnp.int32)
            plsc.addupdate_scatter(den_v.at[0], [d16], ex)
            return c2
        lax.fori_loop(0, _B // 16, _grp, 0)

        def _scale(e_i, c2):
            spl = plsc.bitcast(
                plsc.load_gather(idx_m.at[2], [jnp.broadcast_to(e_i, (16,))]),
                jnp.float32)
            for j in range(_D // 16):
                rows_m[e_i, pl.ds(j * 16, 16)] = (
                    rows_m[e_i, pl.ds(j * 16, 16)] * spl)
            return c2
        lax.fori_loop(0, _B, _scale, 0)

        pltpu.async_copy(rows_m, agg_sh.at[idx_m.at[1]], ssem_m, add=True)

        @pl.when(b + 3 < nb)
        def _():
            pltpu.async_copy(idx_hbm.at[ibase + b + 3], idx_m, isem_m)

    da = pltpu.async_copy(idx_hbm.at[ibase], idx_a, isem_a)
    db = pltpu.async_copy(idx_hbm.at[ibase + 1], idx_b, isem_b)
    pltpu.async_copy(idx_hbm.at[ibase + 2], idx_c, isem_c)
    da.wait()
    pltpu.async_copy(tab_hbm.at[idx_a.at[0]], rows_a, gsem_a)
    db.wait()
    pltpu.async_copy(tab_hbm.at[idx_b.at[0]], rows_b, gsem_b)

    def _triple(q, c):
        _step(3 * q, idx_a, rows_a, gsem_a, isem_a, ssem_a,
              idx_c, rows_c, gsem_c, isem_c, ssem_c)
        _step(3 * q + 1, idx_b, rows_b, gsem_b, isem_b, ssem_b,
              idx_a, rows_a, gsem_a, isem_a, ssem_a)
        _step(3 * q + 2, idx_c, rows_c, gsem_c, isem_c, ssem_c,
              idx_b, rows_b, gsem_b, isem_b, ssem_b)
        return c
    lax.fori_loop(0, nb // 3, _triple, 0)

    pltpu.make_async_copy(rows_a, agg_sh.at[idx_a.at[1]], ssem_a).wait()
    pltpu.make_async_copy(rows_b, agg_sh.at[idx_b.at[1]], ssem_b).wait()
    pltpu.make_async_copy(rows_c, agg_sh.at[idx_c.at[1]], ssem_c).wait()

    pltpu.sync_copy(den_v, den_hbm.at[wid])

    plsc.subcore_barrier()
    pltpu.sync_copy(agg_sh.at[pl.ds(base_row, _RT)],
                    agg_hbm.at[cid, pl.ds(base_row, _RT)])


def _sc_edge(idx_stack, asrc, adst, tab):
    mesh = plsc.VectorSubcoreMesh(core_axis_name="c", subcore_axis_name="s")
    f = pl.kernel(
        _sc_edge_body,
        out_type=(jax.ShapeDtypeStruct((_NC, _N, _D), jnp.float32),
                  jax.ShapeDtypeStruct((_NW, 1, _N), jnp.float32)),
        mesh=mesh,
        scratch_types=[
            pltpu.VMEM((_N,), jnp.float32),
            pltpu.VMEM((_N,), jnp.float32),
            pltpu.VMEM((1, _N), jnp.float32),
            pltpu.VMEM((3, _B), jnp.int32),
            pltpu.VMEM((3, _B), jnp.int32),
            pltpu.VMEM((3, _B), jnp.int32),
            pltpu.VMEM((_B, _D), jnp.float32),
            pltpu.VMEM((_B, _D), jnp.float32),
            pltpu.VMEM((_B, _D), jnp.float32),
            pltpu.VMEM_SHARED((_N, _D), jnp.float32),
            pltpu.SemaphoreType.DMA,
            pltpu.SemaphoreType.DMA,
            pltpu.SemaphoreType.DMA,
            pltpu.SemaphoreType.DMA,
            pltpu.SemaphoreType.DMA,
            pltpu.SemaphoreType.DMA,
            pltpu.SemaphoreType.DMA,
            pltpu.SemaphoreType.DMA,
            pltpu.SemaphoreType.DMA,
        ],
        compiler_params=pltpu.CompilerParams(needs_layout_passes=False),
    )
    return f(idx_stack, asrc, adst, tab)


def _alpha1_body(x_ref, w1, as_ref, ad_ref, os_ref, od_ref):
    v1 = jnp.sum(w1[...] * as_ref[...], axis=1, keepdims=True)
    v2 = jnp.sum(w1[...] * ad_ref[...], axis=1, keepdims=True)
    xv = x_ref[...]
    os_ref[...] = lax.dot_general(v1, xv, (((0,), (1,)), ((), ())),
                                  preferred_element_type=jnp.float32)
    od_ref[...] = lax.dot_general(v2, xv, (((0,), (1,)), ((), ())),
                                  preferred_element_type=jnp.float32)


def _alpha1(x, W1, a_src, a_dst):
    return pl.pallas_call(
        _alpha1_body,
        grid=(1,),
        in_specs=[
            pl.BlockSpec((_N, _D), lambda i: (0, 0)),
            pl.BlockSpec((_D, _HID), lambda i: (0, 0)),
            pl.BlockSpec((1, _HID), lambda i: (0, 0)),
            pl.BlockSpec((1, _HID), lambda i: (0, 0)),
        ],
        out_specs=[
            pl.BlockSpec((1, _N), lambda i: (0, 0)),
            pl.BlockSpec((1, _N), lambda i: (0, 0)),
        ],
        out_shape=[
            jax.ShapeDtypeStruct((1, _N), jnp.float32),
            jax.ShapeDtypeStruct((1, _N), jnp.float32),
        ],
    )(x, W1, a_src, a_dst)


def _dense_body(aggp, dent, w1, b1, w2, a2, y_ref, os_ref, od_ref):
    rows = aggp[0] + aggp[1]
    den = jnp.sum(dent[0], axis=1, keepdims=True)
    den = jnp.where(den > 0.0, den, 1.0)
    rows = rows / den
    z = jnp.maximum(
        lax.dot_general(rows, w1[...], (((1,), (0,)), ((), ())),
                        preferred_element_type=jnp.float32) + b1[...], 0.0)
    yv = lax.dot_general(z, w2[...], (((1,), (0,)), ((), ())),
                         preferred_element_type=jnp.float32)
    y_ref[...] = yv
    av = lax.dot_general(a2[...], yv, (((1,), (1,)), ((), ())),
                         preferred_element_type=jnp.float32)
    os_ref[...] = jnp.broadcast_to(av[0:1, :], (1, 8, _R))
    od_ref[...] = jnp.broadcast_to(av[1:2, :], (1, 8, _R))


def _dense(aggp, dent, W1, b1, W2, a2):
    return pl.pallas_call(
        _dense_body,
        grid=(_G,),
        in_specs=[
            pl.BlockSpec((_NC, _R, _D), lambda i: (0, i, 0)),
            pl.BlockSpec((1, _R, _NW), lambda i: (i, 0, 0)),
            pl.BlockSpec((_D, _HID), lambda i: (0, 0)),
            pl.BlockSpec((1, _HID), lambda i: (0, 0)),
            pl.BlockSpec((_HID, _D), lambda i: (0, 0)),
            pl.BlockSpec((2, _D), lambda i: (0, 0)),
        ],
        out_specs=[
            pl.BlockSpec((_R, _D), lambda i: (i, 0)),
            pl.BlockSpec((1, 8, _R), lambda i: (i, 0, 0)),
            pl.BlockSpec((1, 8, _R), lambda i: (i, 0, 0)),
        ],
        out_shape=[
            jax.ShapeDtypeStruct((_N, _D), jnp.float32),
            jax.ShapeDtypeStruct((_G, 8, _R), jnp.float32),
            jax.ShapeDtypeStruct((_G, 8, _R), jnp.float32),
        ],
    )(aggp, dent, W1, b1, W2, a2)


def _finish_body(aggp, dent, b2, o_ref):
    rows = aggp[0] + aggp[1]
    den = jnp.sum(dent[0], axis=1, keepdims=True)
    den = jnp.where(den > 0.0, den, 1.0)
    rows = rows / den + b2[...]
    m = jnp.max(rows, axis=1, keepdims=True)
    ex = jnp.exp(rows - m)
    s = jnp.sum(ex, axis=1, keepdims=True)
    o_ref[...] = rows - m - jnp.log(s)


def _finish(aggp, dent, b2):
    return pl.pallas_call(
        _finish_body,
        grid=(_G,),
        in_specs=[
            pl.BlockSpec((_NC, _R, _D), lambda i: (0, i, 0)),
            pl.BlockSpec((1, _R, _NW), lambda i: (i, 0, 0)),
            pl.BlockSpec((1, _D), lambda i: (0, 0)),
        ],
        out_specs=pl.BlockSpec((_R, _D), lambda i: (i, 0)),
        out_shape=jax.ShapeDtypeStruct((_N, _D), jnp.float32),
    )(aggp, dent, b2)


def kernel(x, edge_index, W1, att_src1, att_dst1, b1,
           W2, att_src2, att_dst2, b2):
    x = x.astype(jnp.float32)
    loop = jnp.arange(_N, dtype=jnp.int32)
    src = jnp.concatenate([edge_index[0].astype(jnp.int32), loop,
                           jnp.zeros((_E_PAD - _E_TOT,), jnp.int32)])
    dst = jnp.concatenate([edge_index[1].astype(jnp.int32), loop,
                           jnp.zeros((_E_PAD - _E_TOT,), jnp.int32)])
    idx_stack = jnp.stack([src.reshape(_NROWS, _B),
                           dst.reshape(_NROWS, _B),
                           jnp.zeros((_NROWS, _B), jnp.int32)], axis=1)

    asrc1, adst1 = _alpha1(x, W1, att_src1.reshape(1, _HID),
                           att_dst1.reshape(1, _HID))
    agg1, den1 = _sc_edge(idx_stack, asrc1.reshape(_N),
                          adst1.reshape(_N), x)
    den1t = _den_t(den1)
    y, as2, ad2 = _dense(agg1, den1t, W1, b1.reshape(1, _HID), W2,
                         jnp.stack([att_src2, att_dst2]))
    agg2, den2 = _sc_edge(idx_stack, as2[:, 0, :].reshape(_N),
                          ad2[:, 0, :].reshape(_N), y)
    den2t = _den_t(den2)
    return _finish(agg2, den2t, b2.reshape(1, _D))


def _den_t(den):
    return den.reshape(_NW, _G, _R).transpose(1, 2, 0)


# speedup vs baseline: 1.3680x; 1.3680x over previous
import functools

import jax
import jax.numpy as jnp
from jax import lax
from jax.experimental import pallas as pl
from jax.experimental.pallas import tpu as pltpu
from jax.experimental.pallas import tpu_sc as plsc

_N = 10000
_D = 128
_HID = 4096
_E_TOT = 170000
_NC = 2
_NS = 16
_NW = _NC * _NS
_B = 64
_NB0 = 116
_NB1 = 52
_NB = (_NB0 + _NB1) // 2
_T = _B * _NB
_E_PAD = _NW * _T
_NP = 10240
_RT = _NP // _NS
_ZB = 128
_R = 512
_G = _NP // _R


def _sc_edge_body(idx_hbm, src_hbm, asrc_hbm, adst_hbm, tab_hbm,
                  agg_hbm, den_hbm,
                  asrc_v, adst_v, den_v, idx_a, idx_b, ex_a, ex_b,
                  rows_a, rows_b, agg_sh,
                  isem_a, isem_b, gsem_a, gsem_b, ssem_a, ssem_b):
    cid = lax.axis_index("c")
    sid = lax.axis_index("s")
    wid = cid * _NS + sid

    pltpu.sync_copy(asrc_hbm, asrc_v)
    pltpu.sync_copy(adst_hbm, adst_v)

    def _zden(i, c):
        den_v[0, pl.ds(i * 16, 16)] = jnp.zeros((16,), jnp.float32)
        return c
    lax.fori_loop(0, _N // 16, _zden, 0)

    def _zrow(i, c):
        for j in range(_D // 16):
            rows_a[i, pl.ds(j * 16, 16)] = jnp.zeros((16,), jnp.float32)
        return c
    lax.fori_loop(0, _B, _zrow, 0)
    base_row = sid * _RT
    for k in range(_RT // _B):
        pltpu.sync_copy(rows_a, agg_sh.at[pl.ds(base_row + k * _B, _B)])
    plsc.subcore_barrier()

    nb = jnp.where(cid == 0, _NB0, _NB1)
    ibase = jnp.where(cid == 0, sid * _NB0, _NS * _NB0 + sid * _NB1)
    ebase = ibase * _B

    def _step(b, idx_m, ex_m, rows_m, gsem_m, isem_m, ssem_m,
              idx_o, ex_o, rows_o, gsem_o, isem_o, ssem_o):
        @pl.when(b + 1 < nb)
        def _():
            pltpu.make_async_copy(idx_hbm.at[ibase + b + 1], idx_o,
                                  isem_o).wait()

            @pl.when(b >= 1)
            def _():
                pltpu.make_async_copy(rows_o, agg_sh.at[idx_o.at[1]],
                                      ssem_o).wait()
            pltpu.async_copy(tab_hbm.at[idx_o.at[0]], rows_o, gsem_o)

        off = ebase + b * _B

        def _grp(g, c2):
            s16 = idx_m[0, pl.ds(g * 16, 16)]
            d16 = idx_m[1, pl.ds(g * 16, 16)]
            av = plsc.load_gather(asrc_v, [s16])
            dv = plsc.load_gather(adst_v, [d16])
            e = av + dv
            e = jnp.maximum(e, 0.2 * e)
            ex = jnp.exp(e)
            gidx = off + g * 16 + lax.iota(jnp.int32, 16)
            ex = jnp.where(gidx < _E_TOT, ex, 0.0)
            ex_m[pl.ds(g * 16, 16)] = ex
            plsc.addupdate_scatter(den_v.at[0], [d16], ex)
            return c2
        lax.fori_loop(0, _B // 16, _grp, 0)

        pltpu.make_async_copy(tab_hbm.at[idx_m.at[0]], rows_m, gsem_m).wait()

        def _scale(i, c2):
            for u in range(2):
                e_i = 2 * i + u
                spl = plsc.load_gather(ex_m, [jnp.broadcast_to(e_i, (16,))])
                for j in range(_D // 16):
                    rows_m[e_i, pl.ds(j * 16, 16)] = (
                        rows_m[e_i, pl.ds(j * 16, 16)] * spl)
            return c2
        lax.fori_loop(0, _B // 2, _scale, 0)

        pltpu.async_copy(rows_m, agg_sh.at[idx_m.at[1]], ssem_m, add=True)

        @pl.when(b + 2 < nb)
        def _():
            pltpu.async_copy(idx_hbm.at[ibase + b + 2], idx_m, isem_m)

    pltpu.async_copy(idx_hbm.at[ibase], idx_a, isem_a).wait()
    pltpu.async_copy(idx_hbm.at[ibase + 1], idx_b, isem_b)
    pltpu.async_copy(tab_hbm.at[idx_a.at[0]], rows_a, gsem_a)

    def _pair(p, c):
        _step(2 * p, idx_a, ex_a, rows_a, gsem_a, isem_a, ssem_a,
              idx_b, ex_b, rows_b, gsem_b, isem_b, ssem_b)
        _step(2 * p + 1, idx_b, ex_b, rows_b, gsem_b, isem_b, ssem_b,
              idx_a, ex_a, rows_a, gsem_a, isem_a, ssem_a)
        return c
    lax.fori_loop(0, nb // 2, _pair, 0)

    pltpu.make_async_copy(rows_a, agg_sh.at[idx_a.at[1]], ssem_a).wait()
    pltpu.make_async_copy(rows_b, agg_sh.at[idx_b.at[1]], ssem_b).wait()

    pltpu.sync_copy(den_v, den_hbm.at[wid])

    plsc.subcore_barrier()
    pltpu.sync_copy(agg_sh.at[pl.ds(base_row, _RT)],
                    agg_hbm.at[cid, pl.ds(base_row, _RT)])


def _sc_edge(idx_stack, src, asrc, adst, tab):
    mesh = plsc.VectorSubcoreMesh(core_axis_name="c", subcore_axis_name="s")
    f = pl.kernel(
        _sc_edge_body,
        out_type=(jax.ShapeDtypeStruct((_NC, _NP, _D), jnp.float32),
                  jax.ShapeDtypeStruct((_NW, 1, _N), jnp.float32)),
        mesh=mesh,
        scratch_types=[
            pltpu.VMEM((_N,), jnp.float32),
            pltpu.VMEM((_N,), jnp.float32),
            pltpu.VMEM((1, _N), jnp.float32),
            pltpu.VMEM((2, _B), jnp.int32),
            pltpu.VMEM((2, _B), jnp.int32),
            pltpu.VMEM((_B,), jnp.float32),
            pltpu.VMEM((_B,), jnp.float32),
            pltpu.VMEM((_B, _D), jnp.float32),
            pltpu.VMEM((_B, _D), jnp.float32),
            pltpu.VMEM_SHARED((_NP, _D), jnp.float32),
            pltpu.SemaphoreType.DMA,
            pltpu.SemaphoreType.DMA,
            pltpu.SemaphoreType.DMA,
            pltpu.SemaphoreType.DMA,
            pltpu.SemaphoreType.DMA,
            pltpu.SemaphoreType.DMA,
        ],
        compiler_params=pltpu.CompilerParams(needs_layout_passes=False),
    )
    return f(idx_stack, src, asrc, adst, tab)


def _alpha1_body(x_ref, w1, as_ref, ad_ref, os_ref, od_ref):
    v1 = jnp.sum(w1[...] * as_ref[...], axis=1, keepdims=True)
    v2 = jnp.sum(w1[...] * ad_ref[...], axis=1, keepdims=True)
    xv = x_ref[...]
    os_ref[...] = lax.dot_general(v1, xv, (((0,), (1,)), ((), ())),
                                  preferred_element_type=jnp.float32)
    od_ref[...] = lax.dot_general(v2, xv, (((0,), (1,)), ((), ())),
                                  preferred_element_type=jnp.float32)


def _alpha1(x, W1, a_src, a_dst):
    return pl.pallas_call(
        _alpha1_body,
        grid=(1,),
        in_specs=[
            pl.BlockSpec((_N, _D), lambda i: (0, 0)),
            pl.BlockSpec((_D, _HID), lambda i: (0, 0)),
            pl.BlockSpec((1, _HID), lambda i: (0, 0)),
            pl.BlockSpec((1, _HID), lambda i: (0, 0)),
        ],
        out_specs=[
            pl.BlockSpec((1, _N), lambda i: (0, 0)),
            pl.BlockSpec((1, _N), lambda i: (0, 0)),
        ],
        out_shape=[
            jax.ShapeDtypeStruct((1, _N), jnp.float32),
            jax.ShapeDtypeStruct((1, _N), jnp.float32),
        ],
    )(x, W1, a_src, a_dst)


def _dense_body(aggp, dent, w1, b1, w2, a2, y_ref, os_ref, od_ref):
    rows = aggp[0] + aggp[1]
    den = jnp.sum(dent[0], axis=1, keepdims=True)
    den = jnp.where(den > 0.0, den, 1.0)
    rows = rows / den
    z = jnp.maximum(
        lax.dot_general(rows, w1[...], (((1,), (0,)), ((), ())),
                        preferred_element_type=jnp.float32) + b1[...], 0.0)
    yv = lax.dot_general(z, w2[...], (((1,), (0,)), ((), ())),
                         preferred_element_type=jnp.float32)
    y_ref[...] = yv
    av = lax.dot_general(a2[...], yv, (((1,), (1,)), ((), ())),
                         preferred_element_type=jnp.float32)
    os_ref[...] = jnp.broadcast_to(av[0:1, :], (1, 8, _R))
    od_ref[...] = jnp.broadcast_to(av[1:2, :], (1, 8, _R))


def _dense(aggp, dent, W1, b1, W2, a2):
    return pl.pallas_call(
        _dense_body,
        grid=(_G,),
        in_specs=[
            pl.BlockSpec((_NC, _R, _D), lambda i: (0, i, 0)),
            pl.BlockSpec((1, _R, _NW), lambda i: (i, 0, 0)),
            pl.BlockSpec((_D, _HID), lambda i: (0, 0)),
            pl.BlockSpec((1, _HID), lambda i: (0, 0)),
            pl.BlockSpec((_HID, _D), lambda i: (0, 0)),
            pl.BlockSpec((2, _D), lambda i: (0, 0)),
        ],
        out_specs=[
            pl.BlockSpec((_R, _D), lambda i: (i, 0)),
            pl.BlockSpec((1, 8, _R), lambda i: (i, 0, 0)),
            pl.BlockSpec((1, 8, _R), lambda i: (i, 0, 0)),
        ],
        out_shape=[
            jax.ShapeDtypeStruct((_NP, _D), jnp.float32),
            jax.ShapeDtypeStruct((_G, 8, _R), jnp.float32),
            jax.ShapeDtypeStruct((_G, 8, _R), jnp.float32),
        ],
    )(aggp, dent, W1, b1, W2, a2)


def _finish_body(aggp, dent, b2, o_ref):
    rows = aggp[0] + aggp[1]
    den = jnp.sum(dent[0], axis=1, keepdims=True)
    den = jnp.where(den > 0.0, den, 1.0)
    rows = rows / den + b2[...]
    m = jnp.max(rows, axis=1, keepdims=True)
    ex = jnp.exp(rows - m)
    s = jnp.sum(ex, axis=1, keepdims=True)
    o_ref[...] = rows - m - jnp.log(s)


def _finish(aggp, dent, b2):
    return pl.pallas_call(
        _finish_body,
        grid=(_G,),
        in_specs=[
            pl.BlockSpec((_NC, _R, _D), lambda i: (0, i, 0)),
            pl.BlockSpec((1, _R, _NW), lambda i: (i, 0, 0)),
            pl.BlockSpec((1, _D), lambda i: (0, 0)),
        ],
        out_specs=pl.BlockSpec((_R, _D), lambda i: (i, 0)),
        out_shape=jax.ShapeDtypeStruct((_NP, _D), jnp.float32),
    )(aggp, dent, b2)


def kernel(x, edge_index, W1, att_src1, att_dst1, b1,
           W2, att_src2, att_dst2, b2):
    x = x.astype(jnp.float32)
    loop = jnp.arange(_N, dtype=jnp.int32)
    src = jnp.concatenate([edge_index[0].astype(jnp.int32), loop,
                           jnp.zeros((_E_PAD - _E_TOT,), jnp.int32)])
    dst = jnp.concatenate([edge_index[1].astype(jnp.int32), loop,
                           jnp.zeros((_E_PAD - _E_TOT,), jnp.int32)])
    idx_stack = jnp.stack([src.reshape(_NW * _NB, _B),
                           dst.reshape(_NW * _NB, _B)], axis=1)

    asrc1, adst1 = _alpha1(x, W1, att_src1.reshape(1, _HID),
                           att_dst1.reshape(1, _HID))
    agg1, den1 = _sc_edge(idx_stack, src, asrc1.reshape(_N),
                          adst1.reshape(_N), x)
    den1t = _den_t(den1)
    y, as2, ad2 = _dense(agg1, den1t, W1, b1.reshape(1, _HID), W2,
                         jnp.stack([att_src2, att_dst2]))
    agg2, den2 = _sc_edge(idx_stack, src, as2[:, 0, :].reshape(_NP)[:_N],
                          ad2[:, 0, :].reshape(_NP)[:_N], y)
    den2t = _den_t(den2)
    return _finish(agg2, den2t, b2.reshape(1, _D))[:_N]


def _den_t(den):
    den = jnp.pad(den.reshape(_NW, _N), ((0, 0), (0, _NP - _N)))
    return den.reshape(_NW, _G, _R).transpose(1, 2, 0)


# speedup vs baseline: 1.3854x; 1.0127x over previous
import functools

import jax
import jax.numpy as jnp
from jax import lax
from jax.experimental import pallas as pl
from jax.experimental.pallas import tpu as pltpu
from jax.experimental.pallas import tpu_sc as plsc

_N = 10000
_D = 128
_HID = 4096
_E_TOT = 170000
_NC = 2
_NS = 16
_NW = _NC * _NS
_B = 64
_NB0 = 112
_NB1 = 56
_NB = (_NB0 + _NB1) // 2
_T = _B * _NB
_E_PAD = _NW * _T
_NP = 10240
_RT = _NP // _NS
_ZB = 128
_R = 512
_G = _NP // _R


def _sc_edge_body(idx_hbm, src_hbm, asrc_hbm, adst_hbm, tab_hbm,
                  agg_hbm, den_hbm,
                  asrc_v, adst_v, den_v, idx_a, idx_b, ex_a, ex_b,
                  rows_a, rows_b, agg_sh,
                  isem_a, isem_b, gsem_a, gsem_b, ssem_a, ssem_b):
    cid = lax.axis_index("c")
    sid = lax.axis_index("s")
    wid = cid * _NS + sid

    pltpu.sync_copy(asrc_hbm, asrc_v)
    pltpu.sync_copy(adst_hbm, adst_v)

    def _zden(i, c):
        den_v[0, pl.ds(i * 16, 16)] = jnp.zeros((16,), jnp.float32)
        return c
    lax.fori_loop(0, _N // 16, _zden, 0)

    def _zrow(i, c):
        for j in range(_D // 16):
            rows_a[i, pl.ds(j * 16, 16)] = jnp.zeros((16,), jnp.float32)
        return c
    lax.fori_loop(0, _B, _zrow, 0)
    base_row = sid * _RT
    for k in range(_RT // _B):
        pltpu.sync_copy(rows_a, agg_sh.at[pl.ds(base_row + k * _B, _B)])
    plsc.subcore_barrier()

    nb = jnp.where(cid == 0, _NB0, _NB1)
    ibase = jnp.where(cid == 0, sid * _NB0, _NS * _NB0 + sid * _NB1)
    ebase = ibase * _B

    def _step(b, idx_m, ex_m, rows_m, gsem_m, isem_m, ssem_m,
              idx_o, ex_o, rows_o, gsem_o, isem_o, ssem_o):
        @pl.when(b + 1 < nb)
        def _():
            pltpu.make_async_copy(idx_hbm.at[ibase + b + 1], idx_o,
                                  isem_o).wait()

            @pl.when(b >= 1)
            def _():
                pltpu.make_async_copy(rows_o, agg_sh.at[idx_o.at[1]],
                                      ssem_o).wait()
            pltpu.async_copy(tab_hbm.at[idx_o.at[0]], rows_o, gsem_o)

        off = ebase + b * _B

        def _grp(g, c2):
            s16 = idx_m[0, pl.ds(g * 16, 16)]
            d16 = idx_m[1, pl.ds(g * 16, 16)]
            av = plsc.load_gather(asrc_v, [s16])
            dv = plsc.load_gather(adst_v, [d16])
            e = av + dv
            e = jnp.maximum(e, 0.2 * e)
            ex = jnp.exp(e)
            gidx = off + g * 16 + lax.iota(jnp.int32, 16)
            ex = jnp.where(gidx < _E_TOT, ex, 0.0)
            ex_m[pl.ds(g * 16, 16)] = ex
            plsc.addupdate_scatter(den_v.at[0], [d16], ex)
            return c2
        lax.fori_loop(0, _B // 16, _grp, 0)

        pltpu.make_async_copy(tab_hbm.at[idx_m.at[0]], rows_m, gsem_m).wait()

        def _scale(i, c2):
            for u in range(2):
                e_i = 2 * i + u
                spl = plsc.load_gather(ex_m, [jnp.broadcast_to(e_i, (16,))])
                for j in range(_D // 16):
                    rows_m[e_i, pl.ds(j * 16, 16)] = (
                        rows_m[e_i, pl.ds(j * 16, 16)] * spl)
            return c2
        lax.fori_loop(0, _B // 2, _scale, 0)

        pltpu.async_copy(rows_m, agg_sh.at[idx_m.at[1]], ssem_m, add=True)

        @pl.when(b + 2 < nb)
        def _():
            pltpu.async_copy(idx_hbm.at[ibase + b + 2], idx_m, isem_m)

    pltpu.async_copy(idx_hbm.at[ibase], idx_a, isem_a).wait()
    pltpu.async_copy(idx_hbm.at[ibase + 1], idx_b, isem_b)
    pltpu.async_copy(tab_hbm.at[idx_a.at[0]], rows_a, gsem_a)

    def _pair(p, c):
        _step(2 * p, idx_a, ex_a, rows_a, gsem_a, isem_a, ssem_a,
              idx_b, ex_b, rows_b, gsem_b, isem_b, ssem_b)
        _step(2 * p + 1, idx_b, ex_b, rows_b, gsem_b, isem_b, ssem_b,
              idx_a, ex_a, rows_a, gsem_a, isem_a, ssem_a)
        return c
    lax.fori_loop(0, nb // 2, _pair, 0)

    pltpu.make_async_copy(rows_a, agg_sh.at[idx_a.at[1]], ssem_a).wait()
    pltpu.make_async_copy(rows_b, agg_sh.at[idx_b.at[1]], ssem_b).wait()

    pltpu.sync_copy(den_v, den_hbm.at[wid])

    plsc.subcore_barrier()
    pltpu.sync_copy(agg_sh.at[pl.ds(base_row, _RT)],
                    agg_hbm.at[cid, pl.ds(base_row, _RT)])


def _sc_edge(idx_stack, src, asrc, adst, tab):
    mesh = plsc.VectorSubcoreMesh(core_axis_name="c", subcore_axis_name="s")
    f = pl.kernel(
        _sc_edge_body,
        out_type=(jax.ShapeDtypeStruct((_NC, _NP, _D), jnp.float32),
                  jax.ShapeDtypeStruct((_NW, 1, _N), jnp.float32)),
        mesh=mesh,
        scratch_types=[
            pltpu.VMEM((_N,), jnp.float32),
            pltpu.VMEM((_N,), jnp.float32),
            pltpu.VMEM((1, _N), jnp.float32),
            pltpu.VMEM((2, _B), jnp.int32),
            pltpu.VMEM((2, _B), jnp.int32),
            pltpu.VMEM((_B,), jnp.float32),
            pltpu.VMEM((_B,), jnp.float32),
            pltpu.VMEM((_B, _D), jnp.float32),
            pltpu.VMEM((_B, _D), jnp.float32),
            pltpu.VMEM_SHARED((_NP, _D), jnp.float32),
            pltpu.SemaphoreType.DMA,
            pltpu.SemaphoreType.DMA,
            pltpu.SemaphoreType.DMA,
            pltpu.SemaphoreType.DMA,
            pltpu.SemaphoreType.DMA,
            pltpu.SemaphoreType.DMA,
        ],
        compiler_params=pltpu.CompilerParams(needs_layout_passes=False),
    )
    return f(idx_stack, src, asrc, adst, tab)


def _alpha1_body(x_ref, w1, as_ref, ad_ref, os_ref, od_ref):
    v1 = jnp.sum(w1[...] * as_ref[...], axis=1, keepdims=True)
    v2 = jnp.sum(w1[...] * ad_ref[...], axis=1, keepdims=True)
    xv = x_ref[...]
    os_ref[...] = lax.dot_general(v1, xv, (((0,), (1,)), ((), ())),
                                  preferred_element_type=jnp.float32)
    od_ref[...] = lax.dot_general(v2, xv, (((0,), (1,)), ((), ())),
                                  preferred_element_type=jnp.float32)


def _alpha1(x, W1, a_src, a_dst):
    return pl.pallas_call(
        _alpha1_body,
        grid=(1,),
        in_specs=[
            pl.BlockSpec((_N, _D), lambda i: (0, 0)),
            pl.BlockSpec((_D, _HID), lambda i: (0, 0)),
            pl.BlockSpec((1, _HID), lambda i: (0, 0)),
            pl.BlockSpec((1, _HID), lambda i: (0, 0)),
        ],
        out_specs=[
            pl.BlockSpec((1, _N), lambda i: (0, 0)),
            pl.BlockSpec((1, _N), lambda i: (0, 0)),
        ],
        out_shape=[
            jax.ShapeDtypeStruct((1, _N), jnp.float32),
            jax.ShapeDtypeStruct((1, _N), jnp.float32),
        ],
    )(x, W1, a_src, a_dst)


def _dense_body(aggp, dent, w1, b1, w2, a2, y_ref, os_ref, od_ref):
    rows = aggp[0] + aggp[1]
    den = jnp.sum(dent[0], axis=1, keepdims=True)
    den = jnp.where(den > 0.0, den, 1.0)
    rows = rows / den
    z = jnp.maximum(
        lax.dot_general(rows, w1[...], (((1,), (0,)), ((), ())),
                        preferred_element_type=jnp.float32) + b1[...], 0.0)
    yv = lax.dot_general(z, w2[...], (((1,), (0,)), ((), ())),
                         preferred_element_type=jnp.float32)
    y_ref[...] = yv
    av = lax.dot_general(a2[...], yv, (((1,), (1,)), ((), ())),
                         preferred_element_type=jnp.float32)
    os_ref[...] = jnp.broadcast_to(av[0:1, :], (1, 8, _R))
    od_ref[...] = jnp.broadcast_to(av[1:2, :], (1, 8, _R))


def _dense(aggp, dent, W1, b1, W2, a2):
    return pl.pallas_call(
        _dense_body,
        grid=(_G,),
        in_specs=[
            pl.BlockSpec((_NC, _R, _D), lambda i: (0, i, 0)),
            pl.BlockSpec((1, _R, _NW), lambda i: (i, 0, 0)),
            pl.BlockSpec((_D, _HID), lambda i: (0, 0)),
            pl.BlockSpec((1, _HID), lambda i: (0, 0)),
            pl.BlockSpec((_HID, _D), lambda i: (0, 0)),
            pl.BlockSpec((2, _D), lambda i: (0, 0)),
        ],
        out_specs=[
            pl.BlockSpec((_R, _D), lambda i: (i, 0)),
            pl.BlockSpec((1, 8, _R), lambda i: (i, 0, 0)),
            pl.BlockSpec((1, 8, _R), lambda i: (i, 0, 0)),
        ],
        out_shape=[
            jax.ShapeDtypeStruct((_NP, _D), jnp.float32),
            jax.ShapeDtypeStruct((_G, 8, _R), jnp.float32),
            jax.ShapeDtypeStruct((_G, 8, _R), jnp.float32),
        ],
    )(aggp, dent, W1, b1, W2, a2)


def _finish_body(aggp, dent, b2, o_ref):
    rows = aggp[0] + aggp[1]
    den = jnp.sum(dent[0], axis=1, keepdims=True)
    den = jnp.where(den > 0.0, den, 1.0)
    rows = rows / den + b2[...]
    m = jnp.max(rows, axis=1, keepdims=True)
    ex = jnp.exp(rows - m)
    s = jnp.sum(ex, axis=1, keepdims=True)
    o_ref[...] = rows - m - jnp.log(s)


def _finish(aggp, dent, b2):
    return pl.pallas_call(
        _finish_body,
        grid=(_G,),
        in_specs=[
            pl.BlockSpec((_NC, _R, _D), lambda i: (0, i, 0)),
            pl.BlockSpec((1, _R, _NW), lambda i: (i, 0, 0)),
            pl.BlockSpec((1, _D), lambda i: (0, 0)),
        ],
        out_specs=pl.BlockSpec((_R, _D), lambda i: (i, 0)),
        out_shape=jax.ShapeDtypeStruct((_NP, _D), jnp.float32),
    )(aggp, dent, b2)


def kernel(x, edge_index, W1, att_src1, att_dst1, b1,
           W2, att_src2, att_dst2, b2):
    x = x.astype(jnp.float32)
    loop = jnp.arange(_N, dtype=jnp.int32)
    src = jnp.concatenate([edge_index[0].astype(jnp.int32), loop,
                           jnp.zeros((_E_PAD - _E_TOT,), jnp.int32)])
    dst = jnp.concatenate([edge_index[1].astype(jnp.int32), loop,
                           jnp.zeros((_E_PAD - _E_TOT,), jnp.int32)])
    idx_stack = jnp.stack([src.reshape(_NW * _NB, _B),
                           dst.reshape(_NW * _NB, _B)], axis=1)

    asrc1, adst1 = _alpha1(x, W1, att_src1.reshape(1, _HID),
                           att_dst1.reshape(1, _HID))
    agg1, den1 = _sc_edge(idx_stack, src, asrc1.reshape(_N),
                          adst1.reshape(_N), x)
    den1t = _den_t(den1)
    y, as2, ad2 = _dense(agg1, den1t, W1, b1.reshape(1, _HID), W2,
                         jnp.stack([att_src2, att_dst2]))
    agg2, den2 = _sc_edge(idx_stack, src, as2[:, 0, :].reshape(_NP)[:_N],
                          ad2[:, 0, :].reshape(_NP)[:_N], y)
    den2t = _den_t(den2)
    return _finish(agg2, den2t, b2.reshape(1, _D))[:_N]


def _den_t(den):
    den = jnp.pad(den.reshape(_NW, _N), ((0, 0), (0, _NP - _N)))
    return den.reshape(_NW, _G, _R).transpose(1, 2, 0)
